# baseline (speedup 1.0000x reference)
import functools

import jax
import jax.numpy as jnp
from jax import lax
from jax.experimental import pallas as pl
from jax.experimental.pallas import tpu as pltpu
from jax.experimental.pallas import tpu_sc as plsc

_B, _S, _K, _D = 2, 2048, 2, 1024
_E = 8
_TOPK = 2
_DFF = 2048
_N = _B * _S * _K
_R = _N * _TOPK
_TM = 256
_P = _R + _E * _TM
_NT = _P // _TM

_NEG = -3.0e38


def _gate_body(x_ref, gw_ref, out_ref):
    logits = jnp.dot(x_ref[...], gw_ref[...], preferred_element_type=jnp.float32)
    rows = logits.shape[0]
    col = lax.broadcasted_iota(jnp.int32, (rows, _E), 1)
    m1 = jnp.max(logits, axis=1, keepdims=True)
    i1 = jnp.min(jnp.where(logits == m1, col, _E), axis=1, keepdims=True)
    masked = jnp.where(col == i1, _NEG, logits)
    m2 = jnp.max(masked, axis=1, keepdims=True)
    i2 = jnp.min(jnp.where(masked == m2, col, _E), axis=1, keepdims=True)
    w1st = 1.0 / (1.0 + jnp.exp(m2 - m1))
    w2nd = 1.0 - w1st
    out_ref[:, 0:1] = i1.astype(jnp.float32)
    out_ref[:, 1:2] = i2.astype(jnp.float32)
    out_ref[:, 2:3] = w1st
    out_ref[:, 3:4] = w2nd


def _gating(x2d, gate_w):
    blk = 1024
    return pl.pallas_call(
        _gate_body,
        grid=(_N // blk,),
        in_specs=[
            pl.BlockSpec((blk, _D), lambda i: (i, 0)),
            pl.BlockSpec((_D, _E), lambda i: (0, 0)),
        ],
        out_specs=pl.BlockSpec((blk, 4), lambda i: (i, 0)),
        out_shape=jax.ShapeDtypeStruct((_N, 4), jnp.float32),
    )(x2d, gate_w)


def _mm_body(te_ref, xg_ref, w1_ref, w2_ref, rw_ref, y_ref):
    h = jnp.dot(xg_ref[...], w1_ref[0], preferred_element_type=jnp.float32)
    h = h * jax.nn.sigmoid(h)
    y = jnp.dot(h, w2_ref[0], preferred_element_type=jnp.float32)
    y_ref[...] = y * rw_ref[0]


def _grouped_mm(xg, w1, w2, row_w3, tile_expert):
    grid_spec = pltpu.PrefetchScalarGridSpec(
        num_scalar_prefetch=1,
        grid=(_NT,),
        in_specs=[
            pl.BlockSpec((_TM, _D), lambda t, te: (t, 0)),
            pl.BlockSpec((1, _D, _DFF), lambda t, te: (te[t], 0, 0)),
            pl.BlockSpec((1, _DFF, _D), lambda t, te: (te[t], 0, 0)),
            pl.BlockSpec((1, _TM, 1), lambda t, te: (t, 0, 0)),
        ],
        out_specs=pl.BlockSpec((_TM, _D), lambda t, te: (t, 0)),
    )
    return pl.pallas_call(
        _mm_body,
        grid_spec=grid_spec,
        out_shape=jax.ShapeDtypeStruct((_P, _D), jnp.float32),
    )(tile_expert, xg, w1, w2, row_w3)


_NW = 32
_TPW = _N // _NW
_CC = 32


def _make_combine():
    mesh = plsc.VectorSubcoreMesh(core_axis_name="c", subcore_axis_name="s")

    @functools.partial(
        pl.kernel,
        mesh=mesh,
        out_type=jax.ShapeDtypeStruct((_N, _D), jnp.float32),
        scratch_types=[
            pltpu.VMEM((_CC,), jnp.int32),
            pltpu.VMEM((_CC,), jnp.int32),
            pltpu.VMEM((_CC, _D), jnp.float32),
            pltpu.VMEM((_CC, _D), jnp.float32),
            pltpu.VMEM((_CC, _D), jnp.float32),
            pltpu.SemaphoreType.DMA,
        ],
    )
    def combine(y_hbm, inv0_hbm, inv1_hbm, out_hbm, i0_v, i1_v, y0_v, y1_v, o_v, sem):
        wid = lax.axis_index("s") * 2 + lax.axis_index("c")
        base = wid * _TPW

        def chunk_body(ci, carry):
            off = base + ci * _CC
            pltpu.sync_copy(inv0_hbm.at[pl.ds(off, _CC)], i0_v)
            pltpu.sync_copy(inv1_hbm.at[pl.ds(off, _CC)], i1_v)
            cp0 = pltpu.async_copy(y_hbm.at[i0_v], y0_v, sem)
            cp1 = pltpu.async_copy(y_hbm.at[i1_v], y1_v, sem)
            cp0.wait()
            cp1.wait()

            def row_body(i, c2):
                for j in range(_D // 16):
                    sl = pl.ds(j * 16, 16)
                    o_v[i, sl] = y0_v[i, sl] + y1_v[i, sl]
                return c2

            lax.fori_loop(0, _CC, row_body, 0)
            pltpu.sync_copy(o_v, out_hbm.at[pl.ds(off, _CC)])
            return carry

        lax.fori_loop(0, _TPW // _CC, chunk_body, 0)

    return combine


_combine = _make_combine()


def _route(ids_flat, wts_flat):
    e_arange = jnp.arange(_E, dtype=jnp.int32)
    counts = jnp.sum(
        (ids_flat[:, None] == e_arange[None, :]).astype(jnp.int32), axis=0
    )
    cnt_pad = ((counts + _TM - 1) // _TM) * _TM
    cum_pad = jnp.cumsum(cnt_pad)
    start_pad = cum_pad - cnt_pad
    cum = jnp.cumsum(counts)
    starts = cum - counts
    order = jnp.argsort(ids_flat, stable=True)
    e_sorted = ids_flat[order]
    j = jnp.arange(_R, dtype=jnp.int32)
    pos = start_pad[e_sorted] + (j - starts[e_sorted])
    row_token = jnp.zeros((_P,), jnp.int32).at[pos].set(
        (order // _TOPK).astype(jnp.int32)
    )
    row_w = jnp.zeros((_P,), jnp.float32).at[pos].set(wts_flat[order])
    inv = jnp.zeros((_R,), jnp.int32).at[order].set(pos)
    tile_base = jnp.arange(_NT, dtype=jnp.int32) * _TM
    tile_expert = jnp.clip(
        jnp.searchsorted(cum_pad, tile_base, side="right"), 0, _E - 1
    ).astype(jnp.int32)
    return row_token, row_w, inv, tile_expert


def kernel(inputs, gate_w, w1, w2):
    x2d = inputs.reshape(_N, _D)
    g = _gating(x2d, gate_w)
    ids = g[:, :2].astype(jnp.int32)
    wts = g[:, 2:4]
    row_token, row_w, inv, tile_expert = _route(ids.reshape(-1), wts.reshape(-1))
    xg = x2d[row_token]
    y = _grouped_mm(xg, w1, w2, row_w.reshape(_NT, _TM, 1), tile_expert)
    inv2 = inv.reshape(_N, _TOPK)
    out = _combine(y, inv2[:, 0], inv2[:, 1])
    return out.reshape(_B, _S, _K, _D)


# speedup vs baseline: 1.1391x; 1.1391x over previous
import functools

import jax
import jax.numpy as jnp
from jax import lax
from jax.experimental import pallas as pl
from jax.experimental.pallas import tpu as pltpu
from jax.experimental.pallas import tpu_sc as plsc

_B, _S, _K, _D = 2, 2048, 2, 1024
_E = 8
_TOPK = 2
_DFF = 2048
_N = _B * _S * _K
_R = _N * _TOPK
_TM = 256
_P = _R + _E * _TM
_NT = _P // _TM

_NEG = -3.0e38


def _gate_body(x_ref, gw_ref, out_ref):
    logits = jnp.dot(x_ref[...], gw_ref[...], preferred_element_type=jnp.float32)
    rows = logits.shape[0]
    col = lax.broadcasted_iota(jnp.int32, (rows, _E), 1)
    m1 = jnp.max(logits, axis=1, keepdims=True)
    i1 = jnp.min(jnp.where(logits == m1, col, _E), axis=1, keepdims=True)
    masked = jnp.where(col == i1, _NEG, logits)
    m2 = jnp.max(masked, axis=1, keepdims=True)
    i2 = jnp.min(jnp.where(masked == m2, col, _E), axis=1, keepdims=True)
    w1st = 1.0 / (1.0 + jnp.exp(m2 - m1))
    w2nd = 1.0 - w1st
    out_ref[:, 0:1] = i1.astype(jnp.float32)
    out_ref[:, 1:2] = i2.astype(jnp.float32)
    out_ref[:, 2:3] = w1st
    out_ref[:, 3:4] = w2nd


def _gating(x2d, gate_w):
    blk = 1024
    return pl.pallas_call(
        _gate_body,
        grid=(_N // blk,),
        in_specs=[
            pl.BlockSpec((blk, _D), lambda i: (i, 0)),
            pl.BlockSpec((_D, _E), lambda i: (0, 0)),
        ],
        out_specs=pl.BlockSpec((blk, 4), lambda i: (i, 0)),
        out_shape=jax.ShapeDtypeStruct((_N, 4), jnp.float32),
    )(x2d, gate_w)


def _mm_body(te_ref, xg_ref, w1_ref, w2_ref, rw_ref, y_ref):
    h = jnp.dot(xg_ref[...], w1_ref[0], preferred_element_type=jnp.float32)
    h = h * jax.nn.sigmoid(h)
    y = jnp.dot(h, w2_ref[0], preferred_element_type=jnp.float32)
    y_ref[...] = y * rw_ref[0]


def _grouped_mm(xg, w1, w2, row_w3, tile_expert):
    grid_spec = pltpu.PrefetchScalarGridSpec(
        num_scalar_prefetch=1,
        grid=(_NT,),
        in_specs=[
            pl.BlockSpec((_TM, _D), lambda t, te: (t, 0)),
            pl.BlockSpec((1, _D, _DFF), lambda t, te: (te[t], 0, 0)),
            pl.BlockSpec((1, _DFF, _D), lambda t, te: (te[t], 0, 0)),
            pl.BlockSpec((1, _TM, 1), lambda t, te: (t, 0, 0)),
        ],
        out_specs=pl.BlockSpec((_TM, _D), lambda t, te: (t, 0)),
    )
    return pl.pallas_call(
        _mm_body,
        grid_spec=grid_spec,
        out_shape=jax.ShapeDtypeStruct((_P, _D), jnp.float32),
    )(tile_expert, xg, w1, w2, row_w3)


_NW = 32
_TPW = _N // _NW
_CC = 16
_RPW = _P // _NW
_GC = 32


def _make_gather():
    mesh = plsc.VectorSubcoreMesh(core_axis_name="c", subcore_axis_name="s")
    nch = _RPW // _GC

    @functools.partial(
        pl.kernel,
        mesh=mesh,
        out_type=jax.ShapeDtypeStruct((_P, _D), jnp.float32),
        scratch_types=[
            pltpu.VMEM((_GC,), jnp.int32),
            pltpu.VMEM((_GC, _D), jnp.float32),
            pltpu.VMEM((_GC,), jnp.int32),
            pltpu.VMEM((_GC, _D), jnp.float32),
            pltpu.SemaphoreType.DMA,
            pltpu.SemaphoreType.DMA,
        ],
    )
    def gather(x_hbm, rt_hbm, xg_hbm, i_a, b_a, i_b, b_b, sem_a, sem_b):
        wid = lax.axis_index("s") * 2 + lax.axis_index("c")
        base = wid * _RPW
        pltpu.sync_copy(rt_hbm.at[pl.ds(base, _GC)], i_a)
        pltpu.async_copy(x_hbm.at[i_a], b_a, sem_a)

        def body(k, carry):
            c0 = base + 2 * k * _GC
            c1 = c0 + _GC
            pltpu.sync_copy(rt_hbm.at[pl.ds(c1, _GC)], i_b)
            pltpu.async_copy(x_hbm.at[i_b], b_b, sem_b)
            pltpu.make_async_copy(x_hbm.at[i_a], b_a, sem_a).wait()
            pltpu.sync_copy(b_a, xg_hbm.at[pl.ds(c0, _GC)])

            @pl.when(k < nch // 2 - 1)
            def _():
                pltpu.sync_copy(rt_hbm.at[pl.ds(c1 + _GC, _GC)], i_a)
                pltpu.async_copy(x_hbm.at[i_a], b_a, sem_a)

            pltpu.make_async_copy(x_hbm.at[i_b], b_b, sem_b).wait()
            pltpu.sync_copy(b_b, xg_hbm.at[pl.ds(c1, _GC)])
            return carry

        lax.fori_loop(0, nch // 2, body, 0)

    return gather


def _make_combine():
    mesh = plsc.VectorSubcoreMesh(core_axis_name="c", subcore_axis_name="s")
    nch = _TPW // _CC

    @functools.partial(
        pl.kernel,
        mesh=mesh,
        out_type=jax.ShapeDtypeStruct((_N, _D), jnp.float32),
        scratch_types=[
            pltpu.VMEM((_CC,), jnp.int32),
            pltpu.VMEM((_CC,), jnp.int32),
            pltpu.VMEM((_CC, _D), jnp.float32),
            pltpu.VMEM((_CC, _D), jnp.float32),
            pltpu.VMEM((_CC,), jnp.int32),
            pltpu.VMEM((_CC,), jnp.int32),
            pltpu.VMEM((_CC, _D), jnp.float32),
            pltpu.VMEM((_CC, _D), jnp.float32),
            pltpu.VMEM((_CC, _D), jnp.float32),
            pltpu.SemaphoreType.DMA,
            pltpu.SemaphoreType.DMA,
        ],
    )
    def combine(y_hbm, inv0_hbm, inv1_hbm, out_hbm,
                i0_a, i1_a, y0_a, y1_a, i0_b, i1_b, y0_b, y1_b, o_v,
                sem_a, sem_b):
        wid = lax.axis_index("s") * 2 + lax.axis_index("c")
        base = wid * _TPW

        def start(idx0, idx1, dst0, dst1, sem, off):
            pltpu.sync_copy(inv0_hbm.at[pl.ds(off, _CC)], idx0)
            pltpu.sync_copy(inv1_hbm.at[pl.ds(off, _CC)], idx1)
            pltpu.async_copy(y_hbm.at[idx0], dst0, sem)
            pltpu.async_copy(y_hbm.at[idx1], dst1, sem)

        def finish(idx0, idx1, dst0, dst1, sem, off):
            pltpu.make_async_copy(y_hbm.at[idx0], dst0, sem).wait()
            pltpu.make_async_copy(y_hbm.at[idx1], dst1, sem).wait()

            def row_body(i, c2):
                for j in range(_D // 16):
                    sl = pl.ds(j * 16, 16)
                    o_v[i, sl] = dst0[i, sl] + dst1[i, sl]
                return c2

            lax.fori_loop(0, _CC, row_body, 0)
            pltpu.sync_copy(o_v, out_hbm.at[pl.ds(off, _CC)])

        start(i0_a, i1_a, y0_a, y1_a, sem_a, base)

        def body(k, carry):
            c0 = base + 2 * k * _CC
            c1 = c0 + _CC
            start(i0_b, i1_b, y0_b, y1_b, sem_b, c1)
            finish(i0_a, i1_a, y0_a, y1_a, sem_a, c0)

            @pl.when(k < nch // 2 - 1)
            def _():
                start(i0_a, i1_a, y0_a, y1_a, sem_a, c1 + _CC)

            finish(i0_b, i1_b, y0_b, y1_b, sem_b, c1)
            return carry

        lax.fori_loop(0, nch // 2, body, 0)

    return combine


_gather = _make_gather()
_combine = _make_combine()


def _route(ids_flat, wts_flat):
    e_arange = jnp.arange(_E, dtype=jnp.int32)
    counts = jnp.sum(
        (ids_flat[:, None] == e_arange[None, :]).astype(jnp.int32), axis=0
    )
    cnt_pad = ((counts + _TM - 1) // _TM) * _TM
    cum_pad = jnp.cumsum(cnt_pad)
    start_pad = cum_pad - cnt_pad
    cum = jnp.cumsum(counts)
    starts = cum - counts
    order = jnp.argsort(ids_flat, stable=True)
    e_sorted = ids_flat[order]
    j = jnp.arange(_R, dtype=jnp.int32)
    pos = start_pad[e_sorted] + (j - starts[e_sorted])
    row_token = jnp.zeros((_P,), jnp.int32).at[pos].set(
        (order // _TOPK).astype(jnp.int32)
    )
    row_w = jnp.zeros((_P,), jnp.float32).at[pos].set(wts_flat[order])
    inv = jnp.zeros((_R,), jnp.int32).at[order].set(pos)
    tile_base = jnp.arange(_NT, dtype=jnp.int32) * _TM
    tile_expert = jnp.clip(
        jnp.searchsorted(cum_pad, tile_base, side="right"), 0, _E - 1
    ).astype(jnp.int32)
    return row_token, row_w, inv, tile_expert


def kernel(inputs, gate_w, w1, w2):
    x2d = inputs.reshape(_N, _D)
    g = _gating(x2d, gate_w)
    ids = g[:, :2].astype(jnp.int32)
    wts = g[:, 2:4]
    row_token, row_w, inv, tile_expert = _route(ids.reshape(-1), wts.reshape(-1))
    xg = _gather(x2d, row_token)
    y = _grouped_mm(xg, w1, w2, row_w.reshape(_NT, _TM, 1), tile_expert)
    inv2 = inv.reshape(_N, _TOPK)
    out = _combine(y, inv2[:, 0], inv2[:, 1])
    return out.reshape(_B, _S, _K, _D)
